# Initial kernel scaffold; baseline (speedup 1.0000x reference)
import jax
import jax.numpy as jnp
from jax.experimental import pallas as pl


def kernel(scores, beam_scores, token_ids):
    raise NotImplementedError("write your pallas kernel here")


# speedup vs baseline: 1.0390x; 1.0390x over previous
import functools

import jax
import jax.numpy as jnp
from jax.experimental import pallas as pl

NUM_BEAMS = 4
EOS = 2
PEN = 1.2
VOCAB = 100000
B = 128
CUR_LEN = 32
ROWS_BLK = 8
K = 2 * NUM_BEAMS


def _row_kernel(scores_ref, tok_ref, vals_ref, cols_ref, lse_ref):
    x = scores_ref[...]
    tok = tok_ref[...]
    col = jax.lax.broadcasted_iota(jnp.int32, x.shape, 1)

    hit = jnp.zeros(x.shape, dtype=jnp.bool_)
    for t in range(CUR_LEN):
        hit = hit | (col == tok[:, t][:, None])
    pen = jnp.where(x < 0.0, x * PEN, x * (1.0 / PEN))
    x = jnp.where(hit, pen, x)

    m = jnp.max(x, axis=1, keepdims=True)
    s = jnp.sum(jnp.exp(x - m), axis=1, keepdims=True)
    lse_ref[...] = m + jnp.log(s)

    big = jnp.int32(2**31 - 1)
    work = x
    for k in range(K):
        mk = jnp.max(work, axis=1, keepdims=True)
        idx = jnp.min(jnp.where(work == mk, col, big), axis=1, keepdims=True)
        vals_ref[:, k] = mk[:, 0]
        cols_ref[:, k] = idx[:, 0]
        work = jnp.where(col == idx, -jnp.inf, work)


@jax.jit
def kernel(scores, beam_scores, token_ids):
    grid = (B // ROWS_BLK,)
    vals, cols, lse = pl.pallas_call(
        _row_kernel,
        grid=grid,
        in_specs=[
            pl.BlockSpec((ROWS_BLK, VOCAB), lambda i: (i, 0)),
            pl.BlockSpec((ROWS_BLK, CUR_LEN), lambda i: (i, 0)),
        ],
        out_specs=[
            pl.BlockSpec((ROWS_BLK, K), lambda i: (i, 0)),
            pl.BlockSpec((ROWS_BLK, K), lambda i: (i, 0)),
            pl.BlockSpec((ROWS_BLK, 1), lambda i: (i, 0)),
        ],
        out_shape=[
            jax.ShapeDtypeStruct((B, K), jnp.float32),
            jax.ShapeDtypeStruct((B, K), jnp.int32),
            jax.ShapeDtypeStruct((B, 1), jnp.float32),
        ],
    )(scores, token_ids)

    bsz = B // NUM_BEAMS
    logp = vals - lse + beam_scores[:, None]
    cand_v = logp.reshape(bsz, NUM_BEAMS * K)
    cand_t = cols.reshape(bsz, NUM_BEAMS * K)
    beam = jnp.repeat(jnp.arange(NUM_BEAMS, dtype=jnp.int32), K)[None, :]
    cand_g = beam * VOCAB + cand_t

    order = jnp.lexsort((cand_g, -cand_v), axis=-1)[:, :K]
    top_v = jnp.take_along_axis(cand_v, order, axis=1)
    top_t = jnp.take_along_axis(cand_t, order, axis=1)
    top_b = jnp.take_along_axis(jnp.broadcast_to(beam, cand_g.shape), order, axis=1)

    not_eos = top_t != EOS
    keep = not_eos & (jnp.cumsum(not_eos.astype(jnp.int32), axis=1) <= NUM_BEAMS)
    sel = jnp.argsort(jnp.where(keep, 0, 1), axis=1, stable=True)[:, :NUM_BEAMS]
    kept_scores = jnp.take_along_axis(top_v, sel, axis=1)
    kept_tokens = jnp.take_along_axis(top_t, sel, axis=1)
    kept_beams = jnp.take_along_axis(top_b, sel, axis=1)
    return kept_scores, kept_tokens, kept_beams


# speedup vs baseline: 1.6210x; 1.5602x over previous
import jax
import jax.numpy as jnp
from jax import lax
from jax.experimental import pallas as pl
from jax.experimental.pallas import tpu as pltpu
from jax.experimental.pallas import tpu_sc as plsc

NUM_BEAMS = 4
EOS = 2
PEN = 1.2
VOCAB = 100000
B = 128
CUR_LEN = 32
ROWS_BLK = 8
K = 2 * NUM_BEAMS

_NC = 2
_NS = 16
_NW = _NC * _NS
_RPW = B // _NW


def _sc_penalize_body(scores_hbm, tok_hbm, out_hbm, row_v, tok_v):
    wid = lax.axis_index("s") * _NC + lax.axis_index("c")
    for j in range(_RPW):
        r = wid * _RPW + j
        pltpu.sync_copy(scores_hbm.at[r], row_v)
        pltpu.sync_copy(tok_hbm.at[r], tok_v)
        idxs = [tok_v[pl.ds(16 * c, 16)] for c in range(CUR_LEN // 16)]
        vals = [plsc.load_gather(row_v, [ix]) for ix in idxs]
        for ix, g in zip(idxs, vals):
            p = jnp.where(g < 0.0, g * PEN, g * (1.0 / PEN))
            plsc.store_scatter(row_v, [ix], p)
        pltpu.sync_copy(row_v, out_hbm.at[r])


_sc_penalize = pl.kernel(
    _sc_penalize_body,
    out_type=jax.ShapeDtypeStruct((B, VOCAB), jnp.float32),
    mesh=plsc.VectorSubcoreMesh(core_axis_name="c", subcore_axis_name="s"),
    scratch_types=[
        pltpu.VMEM((VOCAB,), jnp.float32),
        pltpu.VMEM((CUR_LEN,), jnp.int32),
    ],
    compiler_params=pltpu.CompilerParams(needs_layout_passes=False),
)


def _row_kernel(scores_ref, vals_ref, cols_ref, lse_ref):
    x = scores_ref[...]
    col = jax.lax.broadcasted_iota(jnp.int32, x.shape, 1)

    m = jnp.max(x, axis=1, keepdims=True)
    s = jnp.sum(jnp.exp(x - m), axis=1, keepdims=True)
    lse_ref[...] = m + jnp.log(s)

    big = jnp.int32(2**31 - 1)
    work = x
    for k in range(K):
        mk = jnp.max(work, axis=1, keepdims=True)
        idx = jnp.min(jnp.where(work == mk, col, big), axis=1, keepdims=True)
        vals_ref[:, k] = mk[:, 0]
        cols_ref[:, k] = idx[:, 0]
        work = jnp.where(col == idx, -jnp.inf, work)


@jax.jit
def kernel(scores, beam_scores, token_ids):
    pscores = _sc_penalize(scores, token_ids)

    grid = (B // ROWS_BLK,)
    vals, cols, lse = pl.pallas_call(
        _row_kernel,
        grid=grid,
        in_specs=[
            pl.BlockSpec((ROWS_BLK, VOCAB), lambda i: (i, 0)),
        ],
        out_specs=[
            pl.BlockSpec((ROWS_BLK, K), lambda i: (i, 0)),
            pl.BlockSpec((ROWS_BLK, K), lambda i: (i, 0)),
            pl.BlockSpec((ROWS_BLK, 1), lambda i: (i, 0)),
        ],
        out_shape=[
            jax.ShapeDtypeStruct((B, K), jnp.float32),
            jax.ShapeDtypeStruct((B, K), jnp.int32),
            jax.ShapeDtypeStruct((B, 1), jnp.float32),
        ],
    )(pscores)

    bsz = B // NUM_BEAMS
    logp = vals - lse + beam_scores[:, None]
    cand_v = logp.reshape(bsz, NUM_BEAMS * K)
    cand_t = cols.reshape(bsz, NUM_BEAMS * K)
    beam = jnp.repeat(jnp.arange(NUM_BEAMS, dtype=jnp.int32), K)[None, :]
    cand_g = beam * VOCAB + cand_t

    order = jnp.lexsort((cand_g, -cand_v), axis=-1)[:, :K]
    top_v = jnp.take_along_axis(cand_v, order, axis=1)
    top_t = jnp.take_along_axis(cand_t, order, axis=1)
    top_b = jnp.take_along_axis(jnp.broadcast_to(beam, cand_g.shape), order, axis=1)

    not_eos = top_t != EOS
    keep = not_eos & (jnp.cumsum(not_eos.astype(jnp.int32), axis=1) <= NUM_BEAMS)
    sel = jnp.argsort(jnp.where(keep, 0, 1), axis=1, stable=True)[:, :NUM_BEAMS]
    kept_scores = jnp.take_along_axis(top_v, sel, axis=1)
    kept_tokens = jnp.take_along_axis(top_t, sel, axis=1)
    kept_beams = jnp.take_along_axis(top_b, sel, axis=1)
    return kept_scores, kept_tokens, kept_beams
